# baseline (speedup 1.0000x reference)
import numpy as np
import jax
import jax.numpy as jnp
from jax.experimental import pallas as pl
from jax.experimental.pallas import tpu as pltpu

_DEC_LO = np.array([-0.12940952255092145, 0.22414386804185735,
                    0.836516303737469, 0.48296291314469025], dtype=np.float32)
_DEC_HI = np.array([-0.48296291314469025, 0.836516303737469,
                    -0.22414386804185735, -0.12940952255092145], dtype=np.float32)

_R = 8
_WIN = _R + 16
_KCONV = 6 * _WIN + 1


def _dwt_mat(n):
    d = np.zeros((n, n), dtype=np.float32)
    for i in range(n // 2):
        for k in range(4):
            d[i, (2 * i + 1 - k) % n] += _DEC_LO[k]
            d[n // 2 + i, (2 * i + 1 - k) % n] += _DEC_HI[k]
    return d


def _resize_mat(n_in, n_out):
    src = np.clip((np.arange(n_out, dtype=np.float64) + 0.5) * (n_in / n_out) - 0.5,
                  0.0, n_in - 1.0)
    i0 = np.floor(src).astype(np.int64)
    i1 = np.minimum(i0 + 1, n_in - 1)
    w = (src - i0).astype(np.float32)
    r = np.zeros((n_out, n_in), dtype=np.float32)
    r[np.arange(n_out), i0] += 1.0 - w
    r[np.arange(n_out), i1] += w
    return r


_D512 = _dwt_mat(512)
_D512T = np.ascontiguousarray(_D512.T)
_D256 = _dwt_mat(256)
_D256T = np.ascontiguousarray(_D256.T)
_RROW = _resize_mat(128, 512)
_RCOLT = np.ascontiguousarray(_resize_mat(128, 512).T)

_O, _RI, _C, _DH, _DW = np.meshgrid(np.arange(16), np.arange(_R), np.arange(2),
                                    np.arange(3), np.arange(3), indexing="ij")
_BIGW_ROWS = (_O * _R + _RI).ravel()
_BIGW_COLS = ((_C * 3 + _DW) * _WIN + _RI + 7 + _DH).ravel()


def _wav_kernel(x_ref, hfw_ref, lfw_ref, bigw_ref,
                d512_ref, d512t_ref, d256_ref, d256t_ref,
                rrow_ref, rcolt_ref, out_ref, ext_ref):
    f32 = jnp.float32
    bf16 = jnp.bfloat16
    i = pl.program_id(0)
    phase = jax.lax.rem(i, 2)
    prev = 1 - phase

    x = x_ref[0, 0]
    j1 = jnp.dot(jnp.dot(d512_ref[...], x, preferred_element_type=f32),
                 d512t_ref[...], preferred_element_type=f32)
    a1 = j1[:256, :256]
    j2 = jnp.dot(jnp.dot(d256_ref[...], a1, preferred_element_type=f32),
                 d256t_ref[...], preferred_element_type=f32)

    def norm(q):
        m = jnp.max(jnp.max(jnp.abs(q), axis=0, keepdims=True), axis=1,
                    keepdims=True)
        return q * (1.0 / m)

    a2n = norm(j2[:128, :128])
    quads = [(0, 0, 128, a2n), (0, 128, 128, norm(j2[:128, 128:])),
             (128, 0, 128, norm(j2[128:, :128])), (128, 128, 128, norm(j2[128:, 128:])),
             (0, 256, 256, norm(j1[:256, 256:])), (256, 0, 256, norm(j1[256:, :256])),
             (256, 256, 256, norm(j1[256:, 256:]))]
    for r, c, s, qn in quads:
        ext_ref[phase, 0, 8 + r:8 + r + s, c:c + s] = (
            qn * hfw_ref[r:r + s, c:c + s]).astype(bf16)

    z = a2n * lfw_ref[...]
    ext_ref[phase, 1, 8:520, :] = jnp.dot(
        jnp.dot(rrow_ref[...], z, preferred_element_type=f32),
        rcolt_ref[...], preferred_element_type=f32).astype(bf16)

    zrow8 = jnp.zeros((8, 512), dtype=bf16)
    for c in range(2):
        ext_ref[phase, c, 0:8, :] = zrow8
        ext_ref[phase, c, 520:528, :] = zrow8

    bigw = bigw_ref[...]
    zcol = jnp.zeros((_WIN, 1), dtype=bf16)
    ones_row = jnp.ones((1, 512), dtype=bf16)
    for r0 in range(0, 512, _R):
        groups = []
        for c in range(2):
            s = ext_ref[prev, c, r0:r0 + _WIN, :]
            groups.append(jnp.concatenate([zcol, s[:, :511]], axis=1))
            groups.append(s)
            groups.append(jnp.concatenate([s[:, 1:], zcol], axis=1))
        taps = jnp.concatenate(groups + [ones_row], axis=0)
        out2 = jnp.dot(bigw, taps, preferred_element_type=f32)
        out_ref[0, :, r0:r0 + _R, :] = out2.reshape(16, _R, 512)


def kernel(x, high_freq_weight, low_freq_weight, conv_w, conv_b):
    b = x.shape[0]
    f32 = jnp.float32

    vals = jnp.broadcast_to(conv_w[:, None], (16, _R, 2, 3, 3)).reshape(-1)
    bigw = jnp.zeros((16 * _R, _KCONV - 1), dtype=f32).at[_BIGW_ROWS, _BIGW_COLS].set(vals)
    bigw = jnp.concatenate([bigw, jnp.repeat(conv_b, _R)[:, None]], axis=1).astype(jnp.bfloat16)

    full = lambda shape: pl.BlockSpec(shape, lambda i: (0,) * len(shape))
    out = pl.pallas_call(
        _wav_kernel,
        grid=(b + 1,),
        in_specs=[
            pl.BlockSpec((1, 1, 512, 512),
                         lambda i: (jnp.minimum(i, b - 1), 0, 0, 0)),
            full((512, 512)), full((128, 128)), full((16 * _R, _KCONV)),
            full((512, 512)), full((512, 512)), full((256, 256)), full((256, 256)),
            full((512, 128)), full((128, 512)),
        ],
        out_specs=pl.BlockSpec((1, 16, 512, 512),
                               lambda i: (jnp.maximum(i - 1, 0), 0, 0, 0)),
        out_shape=jax.ShapeDtypeStruct((b, 16, 512, 512), f32),
        scratch_shapes=[pltpu.VMEM((2, 2, 528, 512), jnp.bfloat16)],
        compiler_params=pltpu.CompilerParams(
            dimension_semantics=("arbitrary",),
            vmem_limit_bytes=56 * 1024 * 1024,
        ),
    )(x, high_freq_weight, low_freq_weight, bigw,
      jnp.asarray(_D512), jnp.asarray(_D512T), jnp.asarray(_D256),
      jnp.asarray(_D256T), jnp.asarray(_RROW), jnp.asarray(_RCOLT))
    return out
